# baseline (speedup 1.0000x reference)
import functools

import jax
import jax.numpy as jnp
from jax import lax
from jax.experimental import pallas as pl
from jax.experimental.pallas import tpu as pltpu
from jax.experimental.pallas import tpu_sc as plsc

N = 10000
E = 320000
D = 128
BN_EPS = 1e-5

NC = 2
NS = 16
NW = NC * NS
CHUNK = 125
SUP = 8
ESUP = SUP * CHUNK
SUP0 = 10
SUP1 = 10
TSUP = NS * (SUP0 + SUP1)
N_PAD = 10240
RPT = N_PAD // NS


def _sc_aggregate(h, src3, dst3):
    mesh = plsc.VectorSubcoreMesh(core_axis_name="c", subcore_axis_name="s")

    @functools.partial(
        pl.kernel,
        out_type=jax.ShapeDtypeStruct((NC, N_PAD, D), jnp.float32),
        mesh=mesh,
        scratch_types=[
            pltpu.MemorySpace.VMEM_SHARED((N_PAD, D), jnp.float32),
            pltpu.MemorySpace.VMEM((2, SUP, CHUNK), jnp.int32),
            pltpu.MemorySpace.VMEM((2, SUP, CHUNK), jnp.int32),
            pltpu.MemorySpace.VMEM((CHUNK, D), jnp.float32),
            pltpu.MemorySpace.VMEM((CHUNK, D), jnp.float32),
            pltpu.SemaphoreType.DMA,
            pltpu.SemaphoreType.DMA,
            pltpu.SemaphoreType.DMA,
            pltpu.SemaphoreType.DMA,
            pltpu.SemaphoreType.DMA,
        ],
    )
    def agg_kernel(h_hbm, src_hbm, dst_hbm, out_hbm, acc, src_v, dst_v,
                   rows0, rows1, sem0, sem1, ssem0, ssem1, isem):
        c = lax.axis_index("c")
        s = lax.axis_index("s")
        rows = (rows0, rows1)
        sems = (sem0, sem1)
        ssems = (ssem0, ssem1)

        base = jnp.where(c == 0, s * SUP0, NS * SUP0 + s * SUP1)
        nsup = jnp.where(c == 0, SUP0, SUP1)

        def gather(idx_slice, b):
            pltpu.async_copy(h_hbm.at[idx_slice], rows[b], sems[b])

        def wait_gather(b):
            pltpu.make_async_copy(h_hbm.at[src_v.at[0, 0]], rows[b],
                                  sems[b]).wait()

        def scatter(idx_slice, b):
            pltpu.async_copy(rows[b], acc.at[idx_slice], ssems[b], add=True)

        def wait_scatter(b):
            pltpu.make_async_copy(rows[b], acc.at[dst_v.at[0, 0]],
                                  ssems[b]).wait()

        def prefetch_idx(gsup, p):
            pltpu.async_copy(src_hbm.at[gsup], src_v.at[p], isem)
            pltpu.async_copy(dst_hbm.at[gsup], dst_v.at[p], isem)

        def wait_idx():
            d = pltpu.make_async_copy(src_hbm.at[0], src_v.at[0], isem)
            d.wait()
            d.wait()

        with jax.named_scope("prime"):
            pltpu.sync_copy(src_hbm.at[base], src_v.at[0])
            pltpu.sync_copy(dst_hbm.at[base], dst_v.at[0])
            prefetch_idx(base + 1, 1)
            gather(src_v.at[0, 0], 0)

        with jax.named_scope("zero_acc"):
            zero = jnp.zeros((16,), jnp.float32)

            def zrow(i, _):
                for j in range(D // 16):
                    rows1[i, pl.ds(j * 16, 16)] = zero
                return 0

            lax.fori_loop(0, CHUNK, zrow, 0)
            for r in range(RPT // CHUNK):
                pltpu.sync_copy(rows1, acc.at[pl.ds(s * RPT + r * CHUNK, CHUNK)])
            rem = RPT - (RPT // CHUNK) * CHUNK
            if rem:
                pltpu.sync_copy(
                    rows1.at[pl.ds(0, rem)],
                    acc.at[pl.ds(s * RPT + (RPT // CHUNK) * CHUNK, rem)])
            plsc.subcore_barrier()

        def substep(gather_slice, dst_slice, b, first):
            if not first:
                wait_scatter(1 - b)
            if gather_slice is not None:
                gather(gather_slice, 1 - b)
            wait_gather(b)
            scatter(dst_slice, b)

        def do_super(sup, p, first_super, last_super):
            for k in range(SUP):
                b = k % 2
                first = first_super and k == 0
                if k < SUP - 1:
                    substep(src_v.at[p, k + 1], dst_v.at[p, k], b, first)
                elif not last_super:
                    wait_idx()
                    substep(src_v.at[1 - p, 0], dst_v.at[p, k], b, first)
                    @pl.when(sup + 2 < nsup)
                    def _():
                        prefetch_idx(base + sup + 2, p)
                else:
                    substep(None, dst_v.at[p, k], b, first)

        with jax.named_scope("edges"):
            do_super(0, 0, True, False)

            def super_body(sup, _):
                do_super(sup, lax.rem(sup, 2), False, False)
                return 0

            lax.fori_loop(1, nsup - 1, super_body, 0)
            do_super(nsup - 1, 1, False, True)
            wait_scatter(1)
        with jax.named_scope("post_barrier"):
            plsc.subcore_barrier()

        with jax.named_scope("copy_out"):
            pltpu.sync_copy(acc.at[pl.ds(s * RPT, RPT)],
                            out_hbm.at[c, pl.ds(s * RPT, RPT)])

    return agg_kernel(h, src3, dst3)


def _tc_mlp(x, p, Wa, ba, g, be, Wb, bb):
    def body(x_ref, p_ref, wa_ref, ba_ref, g_ref, be_ref, wb_ref,
             bb_ref, o_ref):
        h = x_ref[...] + p_ref[0, :N, :] + p_ref[1, :N, :]
        t = lax.dot_general(h, wa_ref[...], (((1,), (1,)), ((), ())),
                            preferred_element_type=jnp.float32) + ba_ref[...]
        mu = jnp.mean(t, axis=0, keepdims=True)
        msq = jnp.mean(t * t, axis=0, keepdims=True)
        var = msq - mu * mu
        t = (t - mu) * lax.rsqrt(var + BN_EPS) * g_ref[...] + be_ref[...]
        t = jnp.maximum(t, 0.0)
        o_ref[...] = lax.dot_general(t, wb_ref[...], (((1,), (1,)), ((), ())),
                                     preferred_element_type=jnp.float32) + bb_ref[...]

    return pl.pallas_call(
        body,
        out_shape=jax.ShapeDtypeStruct((N, D), jnp.float32),
    )(x, p, Wa, ba.reshape(1, D), g.reshape(1, D), be.reshape(1, D),
      Wb, bb.reshape(1, D))


def _layer(h, src3, dst3, Wa, ba, g, be, Wb, bb):
    p = _sc_aggregate(h, src3, dst3)
    return _tc_mlp(h, p, Wa, ba, g, be, Wb, bb)


def kernel(x, edge_index, W1a, b1a, g1, be1, W1b, b1b,
           W2a, b2a, g2, be2, W2b, b2b):
    src = edge_index[0]
    dst = edge_index[1]
    src3 = src.reshape(TSUP, SUP, CHUNK)
    dst3 = dst.reshape(TSUP, SUP, CHUNK)

    h = _layer(x, src3, dst3, W1a, b1a, g1, be1, W1b, b1b)
    h = _layer(h, src3, dst3, W2a, b2a, g2, be2, W2b, b2b)
    return h


# speedup vs baseline: 1.0060x; 1.0060x over previous
import functools

import jax
import jax.numpy as jnp
from jax import lax
from jax.experimental import pallas as pl
from jax.experimental.pallas import tpu as pltpu
from jax.experimental.pallas import tpu_sc as plsc

N = 10000
E = 320000
D = 128
BN_EPS = 1e-5

NC = 2
NS = 16
NW = NC * NS
CHUNK = 128
SUP = 8
ESUP = SUP * CHUNK
SUP0 = 10
SUP1 = 10
TSUP = NS * (SUP0 + SUP1)
E_PAD = TSUP * ESUP
N_PAD = 10240
RPT = N_PAD // NS


def _sc_aggregate(h, src3, dst3):
    mesh = plsc.VectorSubcoreMesh(core_axis_name="c", subcore_axis_name="s")

    @functools.partial(
        pl.kernel,
        out_type=jax.ShapeDtypeStruct((NC, N_PAD, D), jnp.float32),
        mesh=mesh,
        scratch_types=[
            pltpu.MemorySpace.VMEM_SHARED((N_PAD, D), jnp.float32),
            pltpu.MemorySpace.VMEM((2, SUP, CHUNK), jnp.int32),
            pltpu.MemorySpace.VMEM((2, SUP, CHUNK), jnp.int32),
            pltpu.MemorySpace.VMEM((CHUNK, D), jnp.float32),
            pltpu.MemorySpace.VMEM((CHUNK, D), jnp.float32),
            pltpu.SemaphoreType.DMA,
            pltpu.SemaphoreType.DMA,
            pltpu.SemaphoreType.DMA,
            pltpu.SemaphoreType.DMA,
            pltpu.SemaphoreType.DMA,
        ],
    )
    def agg_kernel(h_hbm, src_hbm, dst_hbm, out_hbm, acc, src_v, dst_v,
                   rows0, rows1, sem0, sem1, ssem0, ssem1, isem):
        c = lax.axis_index("c")
        s = lax.axis_index("s")
        rows = (rows0, rows1)
        sems = (sem0, sem1)
        ssems = (ssem0, ssem1)

        base = jnp.where(c == 0, s * SUP0, NS * SUP0 + s * SUP1)
        nsup = jnp.where(c == 0, SUP0, SUP1)

        def gather(idx_slice, b):
            pltpu.async_copy(h_hbm.at[idx_slice], rows[b], sems[b])

        def wait_gather(b):
            pltpu.make_async_copy(h_hbm.at[src_v.at[0, 0]], rows[b],
                                  sems[b]).wait()

        def scatter(idx_slice, b):
            pltpu.async_copy(rows[b], acc.at[idx_slice], ssems[b], add=True)

        def wait_scatter(b):
            pltpu.make_async_copy(rows[b], acc.at[dst_v.at[0, 0]],
                                  ssems[b]).wait()

        def prefetch_idx(gsup, p):
            pltpu.async_copy(src_hbm.at[gsup], src_v.at[p], isem)
            pltpu.async_copy(dst_hbm.at[gsup], dst_v.at[p], isem)

        def wait_idx():
            d = pltpu.make_async_copy(src_hbm.at[0], src_v.at[0], isem)
            d.wait()
            d.wait()

        with jax.named_scope("prime"):
            pltpu.sync_copy(src_hbm.at[base], src_v.at[0])
            pltpu.sync_copy(dst_hbm.at[base], dst_v.at[0])
            prefetch_idx(base + 1, 1)
            gather(src_v.at[0, 0], 0)

        with jax.named_scope("zero_acc"):
            zero = jnp.zeros((16,), jnp.float32)

            def zrow(i, _):
                for j in range(D // 16):
                    rows1[i, pl.ds(j * 16, 16)] = zero
                return 0

            lax.fori_loop(0, CHUNK, zrow, 0)
            for r in range(RPT // CHUNK):
                pltpu.sync_copy(rows1, acc.at[pl.ds(s * RPT + r * CHUNK, CHUNK)])
            plsc.subcore_barrier()

        def substep(gather_slice, dst_slice, b, first):
            if not first:
                wait_scatter(1 - b)
            if gather_slice is not None:
                gather(gather_slice, 1 - b)
            wait_gather(b)
            scatter(dst_slice, b)

        def do_super(sup, p, first_super, last_super):
            for k in range(SUP):
                b = k % 2
                first = first_super and k == 0
                if k < SUP - 1:
                    substep(src_v.at[p, k + 1], dst_v.at[p, k], b, first)
                elif not last_super:
                    wait_idx()
                    substep(src_v.at[1 - p, 0], dst_v.at[p, k], b, first)
                    @pl.when(sup + 2 < nsup)
                    def _():
                        prefetch_idx(base + sup + 2, p)
                else:
                    substep(None, dst_v.at[p, k], b, first)

        with jax.named_scope("edges"):
            do_super(0, 0, True, False)

            def super_body(sup, _):
                do_super(sup, lax.rem(sup, 2), False, False)
                return 0

            lax.fori_loop(1, nsup - 1, super_body, 0)
            do_super(nsup - 1, 1, False, True)
            wait_scatter(1)
        with jax.named_scope("post_barrier"):
            plsc.subcore_barrier()

        with jax.named_scope("copy_out"):
            pltpu.sync_copy(acc.at[pl.ds(s * RPT, RPT)],
                            out_hbm.at[c, pl.ds(s * RPT, RPT)])

    return agg_kernel(h, src3, dst3)


def _tc_mlp(x, p, Wa, ba, g, be, Wb, bb):
    def body(x_ref, p_ref, wa_ref, ba_ref, g_ref, be_ref, wb_ref,
             bb_ref, o_ref):
        h = x_ref[...] + p_ref[0, :N, :] + p_ref[1, :N, :]
        t = lax.dot_general(h, wa_ref[...], (((1,), (1,)), ((), ())),
                            preferred_element_type=jnp.float32) + ba_ref[...]
        mu = jnp.mean(t, axis=0, keepdims=True)
        msq = jnp.mean(t * t, axis=0, keepdims=True)
        var = msq - mu * mu
        t = (t - mu) * lax.rsqrt(var + BN_EPS) * g_ref[...] + be_ref[...]
        t = jnp.maximum(t, 0.0)
        o_ref[...] = lax.dot_general(t, wb_ref[...], (((1,), (1,)), ((), ())),
                                     preferred_element_type=jnp.float32) + bb_ref[...]

    return pl.pallas_call(
        body,
        out_shape=jax.ShapeDtypeStruct((N, D), jnp.float32),
    )(x, p, Wa, ba.reshape(1, D), g.reshape(1, D), be.reshape(1, D),
      Wb, bb.reshape(1, D))


def _layer(h, src3, dst3, Wa, ba, g, be, Wb, bb):
    p = _sc_aggregate(h, src3, dst3)
    return _tc_mlp(h, p, Wa, ba, g, be, Wb, bb)


def kernel(x, edge_index, W1a, b1a, g1, be1, W1b, b1b,
           W2a, b2a, g2, be2, W2b, b2b):
    src = edge_index[0]
    dst = edge_index[1]
    pad = E_PAD - E
    ar = jnp.arange(pad, dtype=jnp.int32)
    pad_src = ar % N
    pad_dst = N + ar % (N_PAD - N)
    src3 = jnp.concatenate([src, pad_src]).reshape(TSUP, SUP, CHUNK)
    dst3 = jnp.concatenate([dst, pad_dst]).reshape(TSUP, SUP, CHUNK)

    h = _layer(x, src3, dst3, W1a, b1a, g1, be1, W1b, b1b)
    h = _layer(h, src3, dst3, W2a, b2a, g2, be2, W2b, b2b)
    return h
